# baseline (speedup 1.0000x reference)
import functools

import jax
import jax.numpy as jnp
from jax import lax
from jax.experimental import pallas as pl
from jax.experimental.pallas import tpu as pltpu
from jax.experimental.pallas import tpu_sc as plsc

_NSEG = 64
_MU = 0.1
_BLK = 16000
_L = 16
_NW = 32


def _sc_counts_kernel(label_hbm, out_hbm, lab_v, hist_v, row_v):
    n = label_hbm.shape[0]
    chunk = n // _NW
    wid = lax.axis_index("s") * 2 + lax.axis_index("c")
    pltpu.sync_copy(label_hbm.at[pl.ds(wid * chunk, chunk)], lab_v)

    zeros16 = jnp.zeros((_L,), jnp.float32)
    for k in range(_L * _NSEG // _L):
        hist_v[pl.ds(k * _L, _L)] = zeros16

    lane_base = lax.iota(jnp.int32, _L) * _NSEG
    ones16 = jnp.ones((_L,), jnp.float32)
    unroll = 5
    steps = chunk // _L // unroll

    def body(j, _):
        for k in range(unroll):
            off = (j * unroll + k) * _L
            lab16 = lab_v[pl.ds(off, _L)]
            plsc.addupdate_scatter(hist_v, [lane_base + lab16], ones16)
        return _

    lax.fori_loop(0, steps, body, None)

    for m in range(_NSEG // _L):
        acc = jnp.zeros((_L,), jnp.float32)
        for lane in range(_L):
            acc = acc + hist_v[pl.ds(lane * _NSEG + m * _L, _L)]
        row_v[pl.ds(m * _L, _L)] = acc
    pltpu.sync_copy(row_v, out_hbm.at[wid])


def _sc_counts(label):
    mesh = plsc.VectorSubcoreMesh(core_axis_name="c", subcore_axis_name="s")
    n = label.shape[0]
    chunk = n // _NW
    fn = functools.partial(
        pl.kernel,
        mesh=mesh,
        out_type=jax.ShapeDtypeStruct((_NW, _NSEG), jnp.float32),
        scratch_types=[
            pltpu.VMEM((chunk,), jnp.int32),
            pltpu.VMEM((_L * _NSEG,), jnp.float32),
            pltpu.VMEM((_NSEG,), jnp.float32),
        ],
        compiler_params=pltpu.CompilerParams(needs_layout_passes=False),
    )(_sc_counts_kernel)
    return fn(label)


def _moments_kernel(label_ref, data_ref, cnt_ref, out_ref, acc_x, acc_sq):
    i = pl.program_id(0)
    nsteps = pl.num_programs(0)

    @pl.when(i == 0)
    def _init():
        acc_x[...] = jnp.zeros_like(acc_x)
        acc_sq[...] = jnp.zeros_like(acc_sq)

    lab = label_ref[0]
    data = data_ref[...]
    seg_ids = lax.broadcasted_iota(jnp.int32, (_NSEG, 1), 0)
    ohf = (lab == seg_ids).astype(jnp.float32)
    ohb = ohf.astype(jnp.bfloat16)

    dn = (((1,), (0,)), ((), ()))
    db = data.astype(jnp.bfloat16)
    acc_x[...] += lax.dot_general(ohb, db, dn,
                                  preferred_element_type=jnp.float32)
    acc_sq[...] += lax.dot_general(ohb, (data * data).astype(jnp.bfloat16), dn,
                                   preferred_element_type=jnp.float32)

    @pl.when(i == nsteps - 1)
    def _combine():
        ones_col = jnp.ones((_NW, 1), jnp.float32)
        c = lax.dot_general(cnt_ref[...], ones_col, (((0,), (0,)), ((), ())),
                            preferred_element_type=jnp.float32)
        safe_c = jnp.maximum(c, 1.0)
        sum_x = acc_x[...]
        ssd = (jnp.sum(acc_sq[...], axis=1, keepdims=True)
               - jnp.sum(sum_x * sum_x, axis=1, keepdims=True) / safe_c)
        loss2 = ssd / safe_c
        css = c * (c * c - 1.0) / 12.0
        var_idx = css / jnp.maximum(c - 1.0, 1.0)
        std_idx = jnp.sqrt(jnp.maximum(var_idx, 0.0))
        min_std = jnp.sqrt(c * (c + 1.0) / 12.0)
        loss1 = (std_idx - min_std) / safe_c
        present = (c > 0.0).astype(jnp.float32)
        out_ref[0, 0] = jnp.sum(present * (_MU * loss1 + loss2))


def kernel(label, data):
    n, d = data.shape
    grid = n // _BLK
    counts = _sc_counts(label)
    out = pl.pallas_call(
        _moments_kernel,
        grid=(grid,),
        in_specs=[
            pl.BlockSpec((1, 1, _BLK), lambda i: (i, 0, 0)),
            pl.BlockSpec((_BLK, d), lambda i: (i, 0)),
            pl.BlockSpec((_NW, _NSEG), lambda i: (0, 0)),
        ],
        out_specs=pl.BlockSpec(memory_space=pltpu.SMEM),
        out_shape=jax.ShapeDtypeStruct((1, 1), jnp.float32),
        scratch_shapes=[
            pltpu.VMEM((_NSEG, d), jnp.float32),
            pltpu.VMEM((_NSEG, d), jnp.float32),
        ],
        compiler_params=pltpu.CompilerParams(
            dimension_semantics=("arbitrary",),
        ),
    )(label.reshape(grid, 1, _BLK), data, counts)
    return out[0, 0]


# speedup vs baseline: 1.1247x; 1.1247x over previous
import functools

import jax
import jax.numpy as jnp
from jax import lax
from jax.experimental import pallas as pl
from jax.experimental.pallas import tpu as pltpu
from jax.experimental.pallas import tpu_sc as plsc

_NSEG = 64
_MU = 0.1
_BLK = 16000
_L = 16
_NW = 32
_BANKS = 5


def _sc_counts_kernel(label_hbm, out_hbm, lab_v, hist_v, row_v):
    n = label_hbm.shape[0]
    chunk = n // _NW
    wid = lax.axis_index("s") * 2 + lax.axis_index("c")
    pltpu.sync_copy(label_hbm.at[pl.ds(wid * chunk, chunk)], lab_v)

    zeros16 = jnp.zeros((_L,), jnp.float32)
    for k in range(_BANKS * _L * _NSEG // _L):
        hist_v[pl.ds(k * _L, _L)] = zeros16

    lane_base = lax.iota(jnp.int32, _L) * _NSEG
    ones16 = jnp.ones((_L,), jnp.float32)
    steps = chunk // _L // _BANKS

    def body(j, carry):
        for k in range(_BANKS):
            off = (j * _BANKS + k) * _L
            lab16 = lab_v[pl.ds(off, _L)]
            plsc.addupdate_scatter(
                hist_v, [lab16 + (lane_base + k * (_L * _NSEG))], ones16)
        return carry

    lax.fori_loop(0, steps, body, None)

    for m in range(_NSEG // _L):
        acc = jnp.zeros((_L,), jnp.float32)
        for b in range(_BANKS):
            for lane in range(_L):
                acc = acc + hist_v[
                    pl.ds(b * (_L * _NSEG) + lane * _NSEG + m * _L, _L)]
        row_v[pl.ds(m * _L, _L)] = acc
    pltpu.sync_copy(row_v, out_hbm.at[wid])


def _sc_counts(label):
    mesh = plsc.VectorSubcoreMesh(core_axis_name="c", subcore_axis_name="s")
    n = label.shape[0]
    chunk = n // _NW
    fn = functools.partial(
        pl.kernel,
        mesh=mesh,
        out_type=jax.ShapeDtypeStruct((_NW, _NSEG), jnp.float32),
        scratch_types=[
            pltpu.VMEM((chunk,), jnp.int32),
            pltpu.VMEM((_BANKS * _L * _NSEG,), jnp.float32),
            pltpu.VMEM((_NSEG,), jnp.float32),
        ],
        compiler_params=pltpu.CompilerParams(needs_layout_passes=False),
    )(_sc_counts_kernel)
    return fn(label)


def _moments_kernel(label_ref, data_ref, out_x, out_sq):
    i = pl.program_id(0)

    @pl.when(i == 0)
    def _init():
        out_x[...] = jnp.zeros_like(out_x)
        out_sq[...] = jnp.zeros_like(out_sq)

    lab = label_ref[0]
    data = data_ref[...]
    seg_ids = lax.broadcasted_iota(jnp.int32, (_NSEG, 1), 0)
    ohf = (lab == seg_ids).astype(jnp.float32)
    ohb = ohf.astype(jnp.bfloat16)

    dn = (((1,), (0,)), ((), ()))
    db = data.astype(jnp.bfloat16)
    out_x[...] += lax.dot_general(ohb, db, dn,
                                  preferred_element_type=jnp.float32)
    out_sq[...] += lax.dot_general(ohb, (data * data).astype(jnp.bfloat16), dn,
                                   preferred_element_type=jnp.float32)


def _combine_kernel(x_ref, sq_ref, cnt_ref, out_ref):
    ones_col = jnp.ones((_NW, 1), jnp.float32)
    c = lax.dot_general(cnt_ref[...], ones_col, (((0,), (0,)), ((), ())),
                        preferred_element_type=jnp.float32)
    safe_c = jnp.maximum(c, 1.0)
    sum_x = x_ref[...]
    ssd = (jnp.sum(sq_ref[...], axis=1, keepdims=True)
           - jnp.sum(sum_x * sum_x, axis=1, keepdims=True) / safe_c)
    loss2 = ssd / safe_c
    css = c * (c * c - 1.0) / 12.0
    var_idx = css / jnp.maximum(c - 1.0, 1.0)
    std_idx = jnp.sqrt(jnp.maximum(var_idx, 0.0))
    min_std = jnp.sqrt(c * (c + 1.0) / 12.0)
    loss1 = (std_idx - min_std) / safe_c
    present = (c > 0.0).astype(jnp.float32)
    out_ref[0, 0] = jnp.sum(present * (_MU * loss1 + loss2))


def kernel(label, data):
    n, d = data.shape
    grid = n // _BLK
    counts = _sc_counts(label)
    acc_x, acc_sq = pl.pallas_call(
        _moments_kernel,
        grid=(grid,),
        in_specs=[
            pl.BlockSpec((1, 1, _BLK), lambda i: (i, 0, 0)),
            pl.BlockSpec((_BLK, d), lambda i: (i, 0)),
        ],
        out_specs=[
            pl.BlockSpec((_NSEG, d), lambda i: (0, 0)),
            pl.BlockSpec((_NSEG, d), lambda i: (0, 0)),
        ],
        out_shape=[
            jax.ShapeDtypeStruct((_NSEG, d), jnp.float32),
            jax.ShapeDtypeStruct((_NSEG, d), jnp.float32),
        ],
        compiler_params=pltpu.CompilerParams(
            dimension_semantics=("arbitrary",),
        ),
    )(label.reshape(grid, 1, _BLK), data)
    out = pl.pallas_call(
        _combine_kernel,
        out_specs=pl.BlockSpec(memory_space=pltpu.SMEM),
        out_shape=jax.ShapeDtypeStruct((1, 1), jnp.float32),
    )(acc_x, acc_sq, counts)
    return out[0, 0]


# speedup vs baseline: 1.1321x; 1.0066x over previous
import functools

import jax
import jax.numpy as jnp
from jax import lax
from jax.experimental import pallas as pl
from jax.experimental.pallas import tpu as pltpu
from jax.experimental.pallas import tpu_sc as plsc

_NSEG = 64
_MU = 0.1
_BLK = 16000
_L = 16
_NW = 32
_BANKS = 1
_UNROLL = 10


def _sc_counts_kernel(label_hbm, out_hbm, lab_v, hist_v, row_v):
    n = label_hbm.shape[0]
    chunk = n // _NW
    wid = lax.axis_index("s") * 2 + lax.axis_index("c")
    pltpu.sync_copy(label_hbm.at[pl.ds(wid * chunk, chunk)], lab_v)

    zeros16 = jnp.zeros((_L,), jnp.float32)
    for k in range(_BANKS * _L * _NSEG // _L):
        hist_v[pl.ds(k * _L, _L)] = zeros16

    lane_base = lax.iota(jnp.int32, _L) * _NSEG
    ones16 = jnp.ones((_L,), jnp.float32)
    steps = chunk // _L // _UNROLL

    def body(j, carry):
        for k in range(_UNROLL):
            off = (j * _UNROLL + k) * _L
            lab16 = lab_v[pl.ds(off, _L)]
            bank = (k % _BANKS) * (_L * _NSEG)
            plsc.addupdate_scatter(
                hist_v, [lab16 + (lane_base + bank)], ones16)
        return carry

    lax.fori_loop(0, steps, body, None)

    for m in range(_NSEG // _L):
        acc = jnp.zeros((_L,), jnp.float32)
        for b in range(_BANKS):
            for lane in range(_L):
                acc = acc + hist_v[
                    pl.ds(b * (_L * _NSEG) + lane * _NSEG + m * _L, _L)]
        row_v[pl.ds(m * _L, _L)] = acc
    pltpu.sync_copy(row_v, out_hbm.at[wid])


def _sc_counts(label):
    mesh = plsc.VectorSubcoreMesh(core_axis_name="c", subcore_axis_name="s")
    n = label.shape[0]
    chunk = n // _NW
    fn = functools.partial(
        pl.kernel,
        mesh=mesh,
        out_type=jax.ShapeDtypeStruct((_NW, _NSEG), jnp.float32),
        scratch_types=[
            pltpu.VMEM((chunk,), jnp.int32),
            pltpu.VMEM((_BANKS * _L * _NSEG,), jnp.float32),
            pltpu.VMEM((_NSEG,), jnp.float32),
        ],
        compiler_params=pltpu.CompilerParams(needs_layout_passes=False),
    )(_sc_counts_kernel)
    return fn(label)


def _moments_kernel(label_ref, data_ref, out_x, out_sq):
    i = pl.program_id(0)

    @pl.when(i == 0)
    def _init():
        out_x[...] = jnp.zeros_like(out_x)
        out_sq[...] = jnp.zeros_like(out_sq)

    lab = label_ref[0]
    data = data_ref[...]
    seg_ids = lax.broadcasted_iota(jnp.int32, (_NSEG, 1), 0)
    ohf = (lab == seg_ids).astype(jnp.float32)
    ohb = ohf.astype(jnp.bfloat16)

    dn = (((1,), (0,)), ((), ()))
    db = data.astype(jnp.bfloat16)
    out_x[...] += lax.dot_general(ohb, db, dn,
                                  preferred_element_type=jnp.float32)
    out_sq[...] += lax.dot_general(ohb, (data * data).astype(jnp.bfloat16), dn,
                                   preferred_element_type=jnp.float32)


def _combine_kernel(x_ref, sq_ref, cnt_ref, out_ref):
    ones_col = jnp.ones((_NW, 1), jnp.float32)
    c = lax.dot_general(cnt_ref[...], ones_col, (((0,), (0,)), ((), ())),
                        preferred_element_type=jnp.float32)
    safe_c = jnp.maximum(c, 1.0)
    sum_x = x_ref[...]
    ssd = (jnp.sum(sq_ref[...], axis=1, keepdims=True)
           - jnp.sum(sum_x * sum_x, axis=1, keepdims=True) / safe_c)
    loss2 = ssd / safe_c
    css = c * (c * c - 1.0) / 12.0
    var_idx = css / jnp.maximum(c - 1.0, 1.0)
    std_idx = jnp.sqrt(jnp.maximum(var_idx, 0.0))
    min_std = jnp.sqrt(c * (c + 1.0) / 12.0)
    loss1 = (std_idx - min_std) / safe_c
    present = (c > 0.0).astype(jnp.float32)
    out_ref[0, 0] = jnp.sum(present * (_MU * loss1 + loss2))


def kernel(label, data):
    n, d = data.shape
    grid = n // _BLK
    counts = _sc_counts(label)
    acc_x, acc_sq = pl.pallas_call(
        _moments_kernel,
        grid=(grid,),
        in_specs=[
            pl.BlockSpec((1, 1, _BLK), lambda i: (i, 0, 0)),
            pl.BlockSpec((_BLK, d), lambda i: (i, 0)),
        ],
        out_specs=[
            pl.BlockSpec((_NSEG, d), lambda i: (0, 0)),
            pl.BlockSpec((_NSEG, d), lambda i: (0, 0)),
        ],
        out_shape=[
            jax.ShapeDtypeStruct((_NSEG, d), jnp.float32),
            jax.ShapeDtypeStruct((_NSEG, d), jnp.float32),
        ],
        compiler_params=pltpu.CompilerParams(
            dimension_semantics=("arbitrary",),
        ),
    )(label.reshape(grid, 1, _BLK), data)
    out = pl.pallas_call(
        _combine_kernel,
        out_specs=pl.BlockSpec(memory_space=pltpu.SMEM),
        out_shape=jax.ShapeDtypeStruct((1, 1), jnp.float32),
    )(acc_x, acc_sq, counts)
    return out[0, 0]
